# baseline (speedup 1.0000x reference)
import jax
import jax.numpy as jnp
from jax.experimental import pallas as pl
from jax.experimental.pallas import tpu as pltpu

BATCH = 128
D_IN = 2048
D_OUT = 8192
K_TOP = 246
K_TILE = 128
N_K = D_IN // K_TILE


def _key_to_float(k):
    bits = jnp.where(k < 0, k ^ jnp.int32(0x7FFFFFFF), k)
    return jax.lax.bitcast_convert_type(bits, jnp.float32)


def _select_mask(acc):
    lo = jnp.full((BATCH, 1), -0x7F800001, jnp.int32)
    hi = jnp.full((BATCH, 1), 0x7F800000, jnp.int32)

    def body(_, carry):
        lo, hi = carry
        floor_avg = (lo & hi) + ((lo ^ hi) >> 1)
        mid = floor_avg + ((lo ^ hi) & 1)
        mid_f = _key_to_float(mid)
        cnt = jnp.sum((acc >= mid_f).astype(jnp.int32), axis=1,
                      keepdims=True)
        pred = cnt >= K_TOP
        lo = jnp.where(pred, mid, lo)
        hi = jnp.where(pred, hi, mid - 1)
        return lo, hi

    lo, hi = jax.lax.fori_loop(0, 32, body, (lo, hi))
    return (acc >= _key_to_float(lo)).astype(jnp.float32)


def _kernel_body(x_ref, w_ref, o_ref):
    i = pl.program_id(0)

    @pl.when(i == 0)
    def _init():
        o_ref[...] = jnp.zeros_like(o_ref)

    o_ref[...] += jnp.dot(x_ref[...], w_ref[...],
                          preferred_element_type=jnp.float32)

    @pl.when(i == N_K - 1)
    def _epilogue():
        o_ref[...] = _select_mask(o_ref[...])


def kernel(image, input_projection):
    return pl.pallas_call(
        _kernel_body,
        grid=(N_K,),
        in_specs=[
            pl.BlockSpec((BATCH, K_TILE), lambda i: (0, i)),
            pl.BlockSpec((K_TILE, D_OUT), lambda i: (i, 0)),
        ],
        out_specs=pl.BlockSpec((BATCH, D_OUT), lambda i: (0, 0)),
        out_shape=jax.ShapeDtypeStruct((BATCH, D_OUT), jnp.float32),
        compiler_params=pltpu.CompilerParams(
            dimension_semantics=("arbitrary",),
        ),
    )(image, input_projection)


# speedup vs baseline: 1.0977x; 1.0977x over previous
import jax
import jax.numpy as jnp
from jax.experimental import pallas as pl
from jax.experimental.pallas import tpu as pltpu

BATCH = 128
D_IN = 2048
D_OUT = 8192
K_TOP = 246
K_TILE = 256
N_K = D_IN // K_TILE


def _key_to_float(k):
    bits = jnp.where(k < 0, k ^ jnp.int32(0x7FFFFFFF), k)
    return jax.lax.bitcast_convert_type(bits, jnp.float32)


def _select_mask(acc):
    lo = jnp.full((BATCH, 1), -0x7F800001, jnp.int32)
    hi = jnp.full((BATCH, 1), 0x7F800000, jnp.int32)

    def body(_, carry):
        lo, hi = carry
        floor_avg = (lo & hi) + ((lo ^ hi) >> 1)
        mid = floor_avg + ((lo ^ hi) & 1)
        mid_f = _key_to_float(mid)
        cnt = jnp.sum((acc >= mid_f).astype(jnp.int32), axis=1,
                      keepdims=True)
        pred = cnt >= K_TOP
        lo = jnp.where(pred, mid, lo)
        hi = jnp.where(pred, hi, mid - 1)
        return lo, hi

    lo, hi = jax.lax.fori_loop(0, 32, body, (lo, hi))
    return (acc >= _key_to_float(lo)).astype(jnp.float32)


def _kernel_body(x_ref, w_ref, o_ref):
    i = pl.program_id(0)

    @pl.when(i == 0)
    def _init():
        o_ref[...] = jnp.zeros_like(o_ref)

    o_ref[...] += jnp.dot(x_ref[...], w_ref[...],
                          preferred_element_type=jnp.float32)

    @pl.when(i == N_K - 1)
    def _epilogue():
        o_ref[...] = _select_mask(o_ref[...])


def kernel(image, input_projection):
    return pl.pallas_call(
        _kernel_body,
        grid=(N_K,),
        in_specs=[
            pl.BlockSpec((BATCH, K_TILE), lambda i: (0, i)),
            pl.BlockSpec((K_TILE, D_OUT), lambda i: (i, 0)),
        ],
        out_specs=pl.BlockSpec((BATCH, D_OUT), lambda i: (0, 0)),
        out_shape=jax.ShapeDtypeStruct((BATCH, D_OUT), jnp.float32),
        compiler_params=pltpu.CompilerParams(
            dimension_semantics=("arbitrary",),
        ),
    )(image, input_projection)
